# Initial kernel scaffold; baseline (speedup 1.0000x reference)
import jax
import jax.numpy as jnp
from jax.experimental import pallas as pl


def kernel(feat, edge_index, W_lin, b_lin, W_ih, W_hh, b_ih, b_hh):
    raise NotImplementedError("write your pallas kernel here")


# speedup vs baseline: 3.3382x; 3.3382x over previous
import functools

import jax
import jax.numpy as jnp
from jax import lax
from jax.experimental import pallas as pl
from jax.experimental.pallas import tpu as pltpu
from jax.experimental.pallas import tpu_sc as plsc

_N = 10000
_NP = 10240
_E = 320000
_D = 128
_STEPS = 5

_NC = 2
_NS = 16
_NW = _NC * _NS
_CHUNK = 128
_EPW = 10240
_EP = _NW * _EPW
_NCH = _EPW // _CHUNK
_RPS = _NP // _NS


def _sc_scatter_body(hw_hbm, src_hbm, dst_hbm, zeros_hbm, out_hbm,
                     src_v, dst_v, rows_v, acc_sh, gsem):
    c = lax.axis_index("c")
    s = lax.axis_index("s")
    wid = s * _NC + c
    row0 = s * _RPS

    pltpu.sync_copy(zeros_hbm, acc_sh.at[pl.ds(row0, _RPS)])
    pltpu.sync_copy(src_hbm.at[wid], src_v)
    pltpu.sync_copy(dst_hbm.at[wid], dst_v)
    plsc.subcore_barrier()

    def chunk(j, carry):
        pltpu.async_copy(hw_hbm.at[src_v.at[j]], rows_v, gsem).wait()
        pltpu.sync_copy(rows_v, acc_sh.at[dst_v.at[j]], add=True)
        return carry

    lax.fori_loop(0, _NCH, chunk, 0)
    plsc.subcore_barrier()

    pltpu.sync_copy(acc_sh.at[pl.ds(row0, _RPS)],
                    out_hbm.at[pl.ds(c * _NP + row0, _RPS)])


@functools.partial(
    pl.kernel,
    out_type=jax.ShapeDtypeStruct((2 * _NP, _D), jnp.float32),
    mesh=plsc.VectorSubcoreMesh(core_axis_name="c", subcore_axis_name="s"),
    scratch_types=[
        pltpu.VMEM((_NCH, _CHUNK), jnp.int32),
        pltpu.VMEM((_NCH, _CHUNK), jnp.int32),
        pltpu.VMEM((_CHUNK, _D), jnp.float32),
        pltpu.VMEM_SHARED((_NP, _D), jnp.float32),
        pltpu.SemaphoreType.DMA,
    ],
    name="ggnn_sc_scatter",
)
def _sc_scatter(hw_hbm, src_hbm, dst_hbm, zeros_hbm, out_hbm,
                src_v, dst_v, rows_v, acc_sh, gsem):
    _sc_scatter_body(hw_hbm, src_hbm, dst_hbm, zeros_hbm, out_hbm,
                     src_v, dst_v, rows_v, acc_sh, gsem)


_BLK = 1024
_DOT = dict(preferred_element_type=jnp.float32)


def _lin_body(h_ref, wlin_ref, blin_ref, hw_ref):
    hw_ref[...] = lax.dot_general(h_ref[...], wlin_ref[...],
                                  (((1,), (1,)), ((), ())), **_DOT) + blin_ref[...]


def _gru_body(a0_ref, a1_ref, h_ref, wih_ref, whh_ref, bih_ref, bhh_ref,
              wlin_ref, blin_ref, h_out, hw_out):
    a = a0_ref[...] + a1_ref[...]
    h = h_ref[...]
    gi = lax.dot_general(a, wih_ref[...], (((1,), (1,)), ((), ())), **_DOT) + bih_ref[...]
    gh = lax.dot_general(h, whh_ref[...], (((1,), (1,)), ((), ())), **_DOT) + bhh_ref[...]
    r = jax.nn.sigmoid(gi[:, :_D] + gh[:, :_D])
    z = jax.nn.sigmoid(gi[:, _D:2 * _D] + gh[:, _D:2 * _D])
    n = jnp.tanh(gi[:, 2 * _D:] + r * gh[:, 2 * _D:])
    hn = (1.0 - z) * n + z * h
    h_out[...] = hn
    hw_out[...] = lax.dot_general(hn, wlin_ref[...],
                                  (((1,), (1,)), ((), ())), **_DOT) + blin_ref[...]


def _full(shape):
    return pl.BlockSpec(shape, lambda i: (0, 0))


def _linear(h, W_lin, b_lin2):
    return pl.pallas_call(
        _lin_body,
        grid=(_NP // _BLK,),
        in_specs=[
            pl.BlockSpec((_BLK, _D), lambda i: (i, 0)),
            _full((_D, _D)),
            _full((1, _D)),
        ],
        out_specs=pl.BlockSpec((_BLK, _D), lambda i: (i, 0)),
        out_shape=jax.ShapeDtypeStruct((_NP, _D), jnp.float32),
    )(h, W_lin, b_lin2)


def _gru_step(acc, h, W_ih, W_hh, b_ih2, b_hh2, W_lin, b_lin2):
    return pl.pallas_call(
        _gru_body,
        grid=(_NP // _BLK,),
        in_specs=[
            pl.BlockSpec((_BLK, _D), lambda i: (i, 0)),
            pl.BlockSpec((_BLK, _D), lambda i: (i + _NP // _BLK, 0)),
            pl.BlockSpec((_BLK, _D), lambda i: (i, 0)),
            _full((3 * _D, _D)),
            _full((3 * _D, _D)),
            _full((1, 3 * _D)),
            _full((1, 3 * _D)),
            _full((_D, _D)),
            _full((1, _D)),
        ],
        out_specs=[
            pl.BlockSpec((_BLK, _D), lambda i: (i, 0)),
            pl.BlockSpec((_BLK, _D), lambda i: (i, 0)),
        ],
        out_shape=[
            jax.ShapeDtypeStruct((_NP, _D), jnp.float32),
            jax.ShapeDtypeStruct((_NP, _D), jnp.float32),
        ],
    )(acc, acc, h, W_ih, W_hh, b_ih2, b_hh2, W_lin, b_lin2)


def kernel(feat, edge_index, W_lin, b_lin, W_ih, W_hh, b_ih, b_hh):
    pad = _EPW - _E // _NW
    src = jnp.pad(edge_index[0].reshape(_NW, _E // _NW),
                  ((0, 0), (0, pad))).reshape(_NW, _NCH, _CHUNK)
    dst = jnp.pad(edge_index[1].reshape(_NW, _E // _NW),
                  ((0, 0), (0, pad)),
                  constant_values=_NP - 1).reshape(_NW, _NCH, _CHUNK)
    b_lin2 = b_lin.reshape(1, _D)
    b_ih2 = b_ih.reshape(1, 3 * _D)
    b_hh2 = b_hh.reshape(1, 3 * _D)

    zeros = jnp.zeros((_RPS, _D), jnp.float32)
    h = jnp.pad(feat, ((0, _NP - _N), (0, 0)))
    hw = _linear(h, W_lin, b_lin2)
    for _ in range(_STEPS):
        acc = _sc_scatter(hw, src, dst, zeros)
        h, hw = _gru_step(acc, h, W_ih, W_hh, b_ih2, b_hh2, W_lin, b_lin2)
    return h[:_N]


# speedup vs baseline: 3.8511x; 1.1536x over previous
import functools

import jax
import jax.numpy as jnp
from jax import lax
from jax.experimental import pallas as pl
from jax.experimental.pallas import tpu as pltpu
from jax.experimental.pallas import tpu_sc as plsc

_N = 10000
_NP = 10240
_E = 320000
_D = 128
_STEPS = 5

_NC = 2
_NS = 16
_NW = _NC * _NS
_CHUNK = 128
_EPW = 10240
_EP = _NW * _EPW
_NCH = _EPW // _CHUNK
_RPS = _NP // _NS
_BCH = 8
_NB2 = _NCH // (2 * _BCH)


def _sc_scatter_body(hw_hbm, src_hbm, dst_hbm, zeros_hbm, out_hbm,
                     srcA, dstA, srcB, dstB, rows0, rows1, acc_sh,
                     g0, g1, iA, iB):
    c = lax.axis_index("c")
    s = lax.axis_index("s")
    wid = s * _NC + c
    row0 = s * _RPS

    pltpu.sync_copy(zeros_hbm, acc_sh.at[pl.ds(row0, _RPS)])
    pltpu.sync_copy(src_hbm.at[wid, pl.ds(0, _BCH)], srcA)
    pltpu.sync_copy(dst_hbm.at[wid, pl.ds(0, _BCH)], dstA)
    pltpu.async_copy(src_hbm.at[wid, pl.ds(_BCH, _BCH)], srcB, iB)
    pltpu.async_copy(dst_hbm.at[wid, pl.ds(_BCH, _BCH)], dstB, iB)
    plsc.subcore_barrier()

    rows = (rows0, rows1)
    gsem = (g0, g1)

    def issue_gather(idx_row, p):
        pltpu.async_copy(hw_hbm.at[idx_row], rows[p], gsem[p])

    def wait_gather(idx_row, p):
        pltpu.make_async_copy(hw_hbm.at[idx_row], rows[p], gsem[p]).wait()

    issue_gather(srcA.at[0], 0)

    def pair(b2, carry):
        for r in range(2 * _BCH):
            p = r % 2
            if r < _BCH:
                cur_src, cur_dst = srcA.at[r], dstA.at[r]
            else:
                cur_src, cur_dst = srcB.at[r - _BCH], dstB.at[r - _BCH]
            if r == _BCH - 1:
                pltpu.make_async_copy(
                    src_hbm.at[wid, pl.ds(0, _BCH)], srcB, iB).wait()
                pltpu.make_async_copy(
                    dst_hbm.at[wid, pl.ds(0, _BCH)], dstB, iB).wait()
                issue_gather(srcB.at[0], 1 - p)
            elif r == 2 * _BCH - 1:
                @pl.when(b2 < _NB2 - 1)
                def _():
                    pltpu.make_async_copy(
                        src_hbm.at[wid, pl.ds(0, _BCH)], srcA, iA).wait()
                    pltpu.make_async_copy(
                        dst_hbm.at[wid, pl.ds(0, _BCH)], dstA, iA).wait()
                    issue_gather(srcA.at[0], 1 - p)
            else:
                nxt = (srcA.at[r + 1] if r + 1 < _BCH
                       else srcB.at[r + 1 - _BCH])
                issue_gather(nxt, 1 - p)
            if r == _BCH:
                @pl.when(b2 < _NB2 - 1)
                def _():
                    blk = (b2 + 1) * 2 * _BCH
                    pltpu.async_copy(
                        src_hbm.at[wid, pl.ds(blk, _BCH)], srcA, iA)
                    pltpu.async_copy(
                        dst_hbm.at[wid, pl.ds(blk, _BCH)], dstA, iA)
            wait_gather(cur_src, p)
            pltpu.sync_copy(rows[p], acc_sh.at[cur_dst], add=True)
        @pl.when(b2 < _NB2 - 1)
        def _():
            blk = (b2 + 1) * 2 * _BCH + _BCH
            pltpu.async_copy(src_hbm.at[wid, pl.ds(blk, _BCH)], srcB, iB)
            pltpu.async_copy(dst_hbm.at[wid, pl.ds(blk, _BCH)], dstB, iB)
        return carry

    lax.fori_loop(0, _NB2, pair, 0)
    plsc.subcore_barrier()

    pltpu.sync_copy(acc_sh.at[pl.ds(row0, _RPS)],
                    out_hbm.at[pl.ds(c * _NP + row0, _RPS)])


@functools.partial(
    pl.kernel,
    out_type=jax.ShapeDtypeStruct((2 * _NP, _D), jnp.float32),
    mesh=plsc.VectorSubcoreMesh(core_axis_name="c", subcore_axis_name="s"),
    scratch_types=[
        pltpu.VMEM((_BCH, _CHUNK), jnp.int32),
        pltpu.VMEM((_BCH, _CHUNK), jnp.int32),
        pltpu.VMEM((_BCH, _CHUNK), jnp.int32),
        pltpu.VMEM((_BCH, _CHUNK), jnp.int32),
        pltpu.VMEM((_CHUNK, _D), jnp.float32),
        pltpu.VMEM((_CHUNK, _D), jnp.float32),
        pltpu.VMEM_SHARED((_NP, _D), jnp.float32),
        pltpu.SemaphoreType.DMA,
        pltpu.SemaphoreType.DMA,
        pltpu.SemaphoreType.DMA,
        pltpu.SemaphoreType.DMA,
    ],
    name="ggnn_sc_scatter",
)
def _sc_scatter(hw_hbm, src_hbm, dst_hbm, zeros_hbm, out_hbm,
                srcA, dstA, srcB, dstB, rows0, rows1, acc_sh,
                g0, g1, iA, iB):
    _sc_scatter_body(hw_hbm, src_hbm, dst_hbm, zeros_hbm, out_hbm,
                     srcA, dstA, srcB, dstB, rows0, rows1, acc_sh,
                     g0, g1, iA, iB)


_BLK = 1024
_DOT = dict(preferred_element_type=jnp.float32)


def _lin_body(h_ref, wlin_ref, blin_ref, hw_ref):
    hw_ref[...] = lax.dot_general(h_ref[...], wlin_ref[...],
                                  (((1,), (1,)), ((), ())), **_DOT) + blin_ref[...]


def _gru_body(a0_ref, a1_ref, h_ref, wih_ref, whh_ref, bih_ref, bhh_ref,
              wlin_ref, blin_ref, h_out, hw_out):
    a = a0_ref[...] + a1_ref[...]
    h = h_ref[...]
    gi = lax.dot_general(a, wih_ref[...], (((1,), (1,)), ((), ())), **_DOT) + bih_ref[...]
    gh = lax.dot_general(h, whh_ref[...], (((1,), (1,)), ((), ())), **_DOT) + bhh_ref[...]
    r = jax.nn.sigmoid(gi[:, :_D] + gh[:, :_D])
    z = jax.nn.sigmoid(gi[:, _D:2 * _D] + gh[:, _D:2 * _D])
    n = jnp.tanh(gi[:, 2 * _D:] + r * gh[:, 2 * _D:])
    hn = (1.0 - z) * n + z * h
    h_out[...] = hn
    hw_out[...] = lax.dot_general(hn, wlin_ref[...],
                                  (((1,), (1,)), ((), ())), **_DOT) + blin_ref[...]


def _full(shape):
    return pl.BlockSpec(shape, lambda i: (0, 0))


def _linear(h, W_lin, b_lin2):
    return pl.pallas_call(
        _lin_body,
        grid=(_NP // _BLK,),
        in_specs=[
            pl.BlockSpec((_BLK, _D), lambda i: (i, 0)),
            _full((_D, _D)),
            _full((1, _D)),
        ],
        out_specs=pl.BlockSpec((_BLK, _D), lambda i: (i, 0)),
        out_shape=jax.ShapeDtypeStruct((_NP, _D), jnp.float32),
    )(h, W_lin, b_lin2)


def _gru_step(acc, h, W_ih, W_hh, b_ih2, b_hh2, W_lin, b_lin2):
    return pl.pallas_call(
        _gru_body,
        grid=(_NP // _BLK,),
        in_specs=[
            pl.BlockSpec((_BLK, _D), lambda i: (i, 0)),
            pl.BlockSpec((_BLK, _D), lambda i: (i + _NP // _BLK, 0)),
            pl.BlockSpec((_BLK, _D), lambda i: (i, 0)),
            _full((3 * _D, _D)),
            _full((3 * _D, _D)),
            _full((1, 3 * _D)),
            _full((1, 3 * _D)),
            _full((_D, _D)),
            _full((1, _D)),
        ],
        out_specs=[
            pl.BlockSpec((_BLK, _D), lambda i: (i, 0)),
            pl.BlockSpec((_BLK, _D), lambda i: (i, 0)),
        ],
        out_shape=[
            jax.ShapeDtypeStruct((_NP, _D), jnp.float32),
            jax.ShapeDtypeStruct((_NP, _D), jnp.float32),
        ],
    )(acc, acc, h, W_ih, W_hh, b_ih2, b_hh2, W_lin, b_lin2)


def kernel(feat, edge_index, W_lin, b_lin, W_ih, W_hh, b_ih, b_hh):
    pad = _EPW - _E // _NW
    src = jnp.pad(edge_index[0].reshape(_NW, _E // _NW),
                  ((0, 0), (0, pad))).reshape(_NW, _NCH, _CHUNK)
    dst = jnp.pad(edge_index[1].reshape(_NW, _E // _NW),
                  ((0, 0), (0, pad)),
                  constant_values=_NP - 1).reshape(_NW, _NCH, _CHUNK)
    b_lin2 = b_lin.reshape(1, _D)
    b_ih2 = b_ih.reshape(1, 3 * _D)
    b_hh2 = b_hh.reshape(1, 3 * _D)

    zeros = jnp.zeros((_RPS, _D), jnp.float32)
    h = jnp.pad(feat, ((0, _NP - _N), (0, 0)))
    hw = _linear(h, W_lin, b_lin2)
    for _ in range(_STEPS):
        acc = _sc_scatter(hw, src, dst, zeros)
        h, hw = _gru_step(acc, h, W_ih, W_hh, b_ih2, b_hh2, W_lin, b_lin2)
    return h[:_N]
